# baseline (speedup 1.0000x reference)
import functools

import jax
import jax.numpy as jnp
from jax import lax
from jax.experimental import pallas as pl
from jax.experimental.pallas import tpu as pltpu
from jax.experimental.pallas import tpu_sc as plsc

N = 50000
D = 256
DC = 16
NCHUNK = D // DC
NCORE = 2
NSUB = 16
NPAD = 51200
RPT = NPAD // NSUB
KPT = RPT // 128
ZROWS = 125
BLK = 1000
NBLK = N // BLK


def _sc_body(x_hbm, idx_hbm, out_hbm, table, xv, idxv, zbuf):
    c = lax.axis_index("c")
    s = lax.axis_index("s")
    row0 = s * RPT

    pltpu.sync_copy(idx_hbm.at[s], idxv)

    def _zb(k, _):
        i = k // (DC // 16)
        j = (k % (DC // 16)) * 16
        zbuf[i, pl.ds(j, 16)] = jnp.zeros((16,), jnp.float32)
        return _
    lax.fori_loop(0, ZROWS * (DC // 16), _zb, None)

    def _chunk(cc, _):
        chunk = c * (NCHUNK // NCORE) + cc

        def _z(z, _):
            pltpu.sync_copy(
                zbuf, table.at[pl.ds(s * (N // NSUB) + z * ZROWS, ZROWS), :])
            return _
        lax.fori_loop(0, (N // NSUB) // ZROWS, _z, None)
        plsc.subcore_barrier()

        pltpu.sync_copy(x_hbm.at[chunk, pl.ds(row0, RPT), :], xv)

        def _scat(j, _):
            pltpu.sync_copy(xv.at[pl.ds(j * 128, 128), :],
                            table.at[idxv.at[j]], add=True)
            return _
        lax.fori_loop(0, KPT, _scat, None)
        plsc.subcore_barrier()

        def _gath(j, _):
            pltpu.sync_copy(table.at[idxv.at[j]],
                            xv.at[pl.ds(j * 128, 128), :])
            return _
        lax.fori_loop(0, KPT, _gath, None)

        pltpu.sync_copy(xv, out_hbm.at[chunk, pl.ds(row0, RPT), :])
        plsc.subcore_barrier()
        return _

    lax.fori_loop(0, NCHUNK // NCORE, _chunk, None)


def _sc_gather_scatter(x_sc, idx3d):
    mesh = plsc.VectorSubcoreMesh(core_axis_name="c", subcore_axis_name="s",
                                  num_cores=NCORE, num_subcores=NSUB)
    f = pl.kernel(
        _sc_body,
        out_type=jax.ShapeDtypeStruct((NCHUNK, NPAD, DC), jnp.float32),
        mesh=mesh,
        scratch_types=[
            pltpu.VMEM_SHARED((N, DC), jnp.float32),
            pltpu.VMEM((RPT, DC), jnp.float32),
            pltpu.VMEM((KPT, 128), jnp.int32),
            pltpu.VMEM((ZROWS, DC), jnp.float32),
        ],
        compiler_params=pltpu.CompilerParams(use_tc_tiling_on_sc=False),
    )
    return f(x_sc, idx3d)


def _mlp1_body(eps_ref, x_ref, g_ref, w1_ref, y_ref, ssum_ref, ssq_ref):
    i = pl.program_id(0)
    eps1 = 1.0 + eps_ref[0, 0]
    h = x_ref[...] * eps1 + g_ref[...]
    y = lax.dot_general(h, w1_ref[...], (((1,), (1,)), ((), ())),
                        preferred_element_type=jnp.float32,
                        precision=lax.Precision.HIGHEST)
    y_ref[...] = y
    s = jnp.sum(y, axis=0, keepdims=True)
    q = jnp.sum(y * y, axis=0, keepdims=True)

    @pl.when(i == 0)
    def _():
        ssum_ref[...] = s
        ssq_ref[...] = q

    @pl.when(i != 0)
    def _():
        ssum_ref[...] += s
        ssq_ref[...] += q


def _mlp1(eps, x, gpad, w1):
    return pl.pallas_call(
        _mlp1_body,
        grid=(NBLK,),
        in_specs=[
            pl.BlockSpec((1, 1), lambda i: (0, 0)),
            pl.BlockSpec((BLK, D), lambda i: (i, 0)),
            pl.BlockSpec((BLK, D), lambda i: (i, 0)),
            pl.BlockSpec((D, D), lambda i: (0, 0)),
        ],
        out_specs=[
            pl.BlockSpec((BLK, D), lambda i: (i, 0)),
            pl.BlockSpec((1, D), lambda i: (0, 0)),
            pl.BlockSpec((1, D), lambda i: (0, 0)),
        ],
        out_shape=[
            jax.ShapeDtypeStruct((N, D), jnp.float32),
            jax.ShapeDtypeStruct((1, D), jnp.float32),
            jax.ShapeDtypeStruct((1, D), jnp.float32),
        ],
        compiler_params=pltpu.CompilerParams(
            dimension_semantics=("arbitrary",)),
    )(eps, x, gpad, w1)


def _mlp2_body(y_ref, ssum_ref, ssq_ref, gamma_ref, beta_ref, w2_ref, o_ref):
    inv_n = 1.0 / N
    mu = ssum_ref[...] * inv_n
    var = ssq_ref[...] * inv_n - mu * mu
    scale = gamma_ref[...] * lax.rsqrt(var + 1e-5)
    shift = beta_ref[...] - mu * scale
    z = jnp.maximum(y_ref[...] * scale + shift, 0.0)
    o_ref[...] = lax.dot_general(z, w2_ref[...], (((1,), (1,)), ((), ())),
                                 preferred_element_type=jnp.float32,
                                 precision=lax.Precision.HIGHEST)


def _mlp2(y, ssum, ssq, gamma, beta, w2):
    return pl.pallas_call(
        _mlp2_body,
        grid=(NBLK,),
        in_specs=[
            pl.BlockSpec((BLK, D), lambda i: (i, 0)),
            pl.BlockSpec((1, D), lambda i: (0, 0)),
            pl.BlockSpec((1, D), lambda i: (0, 0)),
            pl.BlockSpec((1, D), lambda i: (0, 0)),
            pl.BlockSpec((1, D), lambda i: (0, 0)),
            pl.BlockSpec((D, D), lambda i: (0, 0)),
        ],
        out_specs=pl.BlockSpec((BLK, D), lambda i: (i, 0)),
        out_shape=jax.ShapeDtypeStruct((N, D), jnp.float32),
        compiler_params=pltpu.CompilerParams(
            dimension_semantics=("arbitrary",)),
    )(y, ssum, ssq, gamma, beta, w2)


def kernel(x, index_add, eps_param, W1, gamma, beta, W2):
    idx = index_add.astype(jnp.int32)
    xpad = jnp.pad(x, ((0, NPAD - N), (0, 0)))
    x_sc = xpad.reshape(NPAD, NCHUNK, DC).transpose(1, 0, 2)
    pad_idx = (jnp.arange(NPAD - N, dtype=jnp.int32) * 41) % N
    idx3d = jnp.concatenate([idx, pad_idx]).reshape(NSUB, KPT, 128)

    g_sc = _sc_gather_scatter(x_sc, idx3d)
    gpad = g_sc.transpose(1, 0, 2).reshape(NPAD, D)

    eps = eps_param.reshape(1, 1)
    y, ssum, ssq = _mlp1(eps, x, gpad, W1)
    out = _mlp2(y, ssum, ssq, gamma.reshape(1, D), beta.reshape(1, D), W2)
    return out


# speedup vs baseline: 3.9187x; 3.9187x over previous
import functools

import jax
import jax.numpy as jnp
from jax import lax
from jax.experimental import pallas as pl
from jax.experimental.pallas import tpu as pltpu
from jax.experimental.pallas import tpu_sc as plsc

N = 50000
D = 256
DC = 32
NCHUNK = D // DC
NCORE = 2
NSUB = 16
RPT = 3200
KPT = RPT // 128
RPT_L = N - 15 * RPT
KPT_L = 16
NIDX = NSUB * KPT * 128
ZROWS = 625
BLK = 5000
NBLK = N // BLK


def _sc_body(x_hbm, idx_hbm, out_hbm, table, xv, idxv, zbuf, sem, sem2):
    c = lax.axis_index("c")
    s = lax.axis_index("s")
    row0 = s * RPT

    pltpu.sync_copy(idx_hbm.at[s], idxv)

    def _zb(k, _):
        zbuf[k, :] = jnp.zeros((DC,), jnp.bfloat16)
        return _
    lax.fori_loop(0, ZROWS, _zb, None)

    def _chunk(cc, _):
        chunk = c * (NCHUNK // NCORE) + cc
        col = chunk * DC

        @pl.when(s < NSUB - 1)
        def _():
            din = pltpu.async_copy(
                x_hbm.at[pl.ds(row0, RPT), pl.ds(col, DC)], xv, sem2)

            def _z(z, _):
                pltpu.sync_copy(
                    zbuf, table.at[pl.ds((s * 5 + z) * ZROWS, ZROWS), :])
                return _
            lax.fori_loop(0, 5, _z, None)
            din.wait()

        @pl.when(s == NSUB - 1)
        def _():
            din = pltpu.async_copy(
                x_hbm.at[pl.ds(row0, RPT_L), pl.ds(col, DC)],
                xv.at[pl.ds(0, RPT_L), :], sem2)

            def _z(z, _):
                pltpu.sync_copy(
                    zbuf, table.at[pl.ds((s * 5 + z) * ZROWS, ZROWS), :])
                return _
            lax.fori_loop(0, 5, _z, None)

            def _zt(k, _):
                xv[RPT_L + k, :] = jnp.zeros((DC,), jnp.bfloat16)
                return _
            lax.fori_loop(0, KPT_L * 128 - RPT_L, _zt, None)
            din.wait()

        plsc.subcore_barrier()

        def _scat(j, _):
            pltpu.async_copy(xv.at[pl.ds(j * 128, 128), :],
                             table.at[idxv.at[j]], sem, add=True)
            return _

        @pl.when(s < NSUB - 1)
        def _():
            lax.fori_loop(0, KPT, _scat, None)
            pltpu.make_async_copy(
                x_hbm.at[pl.ds(0, RPT), pl.ds(0, DC)], xv, sem).wait()

        @pl.when(s == NSUB - 1)
        def _():
            lax.fori_loop(0, KPT_L, _scat, None)
            pltpu.make_async_copy(
                x_hbm.at[pl.ds(0, KPT_L * 128), pl.ds(0, DC)],
                xv.at[pl.ds(0, KPT_L * 128), :], sem).wait()

        plsc.subcore_barrier()

        def _gath(j, _):
            pltpu.async_copy(table.at[idxv.at[j]],
                             xv.at[pl.ds(j * 128, 128), :], sem)
            return _

        @pl.when(s < NSUB - 1)
        def _():
            lax.fori_loop(0, KPT, _gath, None)
            pltpu.make_async_copy(
                x_hbm.at[pl.ds(0, RPT), pl.ds(0, DC)], xv, sem).wait()
            pltpu.sync_copy(xv, out_hbm.at[pl.ds(row0, RPT), pl.ds(col, DC)])

        @pl.when(s == NSUB - 1)
        def _():
            lax.fori_loop(0, KPT_L, _gath, None)
            pltpu.make_async_copy(
                x_hbm.at[pl.ds(0, KPT_L * 128), pl.ds(0, DC)],
                xv.at[pl.ds(0, KPT_L * 128), :], sem).wait()
            pltpu.sync_copy(xv.at[pl.ds(0, RPT_L), :],
                            out_hbm.at[pl.ds(row0, RPT_L), pl.ds(col, DC)])

        plsc.subcore_barrier()
        return _

    lax.fori_loop(0, NCHUNK // NCORE, _chunk, None)


def _sc_gather_scatter(x, idx3d):
    mesh = plsc.VectorSubcoreMesh(core_axis_name="c", subcore_axis_name="s",
                                  num_cores=NCORE, num_subcores=NSUB)
    f = pl.kernel(
        _sc_body,
        out_type=jax.ShapeDtypeStruct((N, D), jnp.bfloat16),
        mesh=mesh,
        scratch_types=[
            pltpu.VMEM_SHARED((N, DC), jnp.bfloat16),
            pltpu.VMEM((RPT, DC), jnp.bfloat16),
            pltpu.VMEM((KPT, 128), jnp.int32),
            pltpu.VMEM((ZROWS, DC), jnp.bfloat16),
            pltpu.SemaphoreType.DMA,
            pltpu.SemaphoreType.DMA,
        ],
        compiler_params=pltpu.CompilerParams(use_tc_tiling_on_sc=False),
    )
    return f(x, idx3d)


def _mlp1_body(eps_ref, x_ref, g_ref, w1_ref, y_ref, ssum_ref, ssq_ref):
    i = pl.program_id(0)
    eps1 = 1.0 + eps_ref[0, 0]
    h = x_ref[...] * eps1 + g_ref[...].astype(jnp.float32)
    y = lax.dot_general(h, w1_ref[...], (((1,), (1,)), ((), ())),
                        preferred_element_type=jnp.float32,
                        precision=lax.Precision.DEFAULT)
    y_ref[...] = y.astype(jnp.bfloat16)
    s = jnp.sum(y, axis=0, keepdims=True)
    q = jnp.sum(y * y, axis=0, keepdims=True)

    @pl.when(i == 0)
    def _():
        ssum_ref[...] = s
        ssq_ref[...] = q

    @pl.when(i != 0)
    def _():
        ssum_ref[...] += s
        ssq_ref[...] += q


def _mlp1(eps, x, g, w1):
    return pl.pallas_call(
        _mlp1_body,
        grid=(NBLK,),
        in_specs=[
            pl.BlockSpec((1, 1), lambda i: (0, 0)),
            pl.BlockSpec((BLK, D), lambda i: (i, 0)),
            pl.BlockSpec((BLK, D), lambda i: (i, 0)),
            pl.BlockSpec((D, D), lambda i: (0, 0)),
        ],
        out_specs=[
            pl.BlockSpec((BLK, D), lambda i: (i, 0)),
            pl.BlockSpec((1, D), lambda i: (0, 0)),
            pl.BlockSpec((1, D), lambda i: (0, 0)),
        ],
        out_shape=[
            jax.ShapeDtypeStruct((N, D), jnp.bfloat16),
            jax.ShapeDtypeStruct((1, D), jnp.float32),
            jax.ShapeDtypeStruct((1, D), jnp.float32),
        ],
        compiler_params=pltpu.CompilerParams(
            dimension_semantics=("arbitrary",)),
    )(eps, x, g, w1)


def _mlp2_body(y_ref, ssum_ref, ssq_ref, gamma_ref, beta_ref, w2_ref, o_ref):
    inv_n = 1.0 / N
    mu = ssum_ref[...] * inv_n
    var = ssq_ref[...] * inv_n - mu * mu
    scale = gamma_ref[...] * lax.rsqrt(var + 1e-5)
    shift = beta_ref[...] - mu * scale
    z = jnp.maximum(y_ref[...].astype(jnp.float32) * scale + shift, 0.0)
    o_ref[...] = lax.dot_general(z, w2_ref[...], (((1,), (1,)), ((), ())),
                                 preferred_element_type=jnp.float32,
                                 precision=lax.Precision.DEFAULT)


def _mlp2(y, ssum, ssq, gamma, beta, w2):
    return pl.pallas_call(
        _mlp2_body,
        grid=(NBLK,),
        in_specs=[
            pl.BlockSpec((BLK, D), lambda i: (i, 0)),
            pl.BlockSpec((1, D), lambda i: (0, 0)),
            pl.BlockSpec((1, D), lambda i: (0, 0)),
            pl.BlockSpec((1, D), lambda i: (0, 0)),
            pl.BlockSpec((1, D), lambda i: (0, 0)),
            pl.BlockSpec((D, D), lambda i: (0, 0)),
        ],
        out_specs=pl.BlockSpec((BLK, D), lambda i: (i, 0)),
        out_shape=jax.ShapeDtypeStruct((N, D), jnp.float32),
        compiler_params=pltpu.CompilerParams(
            dimension_semantics=("arbitrary",)),
    )(y, ssum, ssq, gamma, beta, w2)


def kernel(x, index_add, eps_param, W1, gamma, beta, W2):
    idx = index_add.astype(jnp.int32)
    pad_idx = (jnp.arange(NIDX - N, dtype=jnp.int32) * 41) % N
    idx3d = jnp.concatenate([idx, pad_idx]).reshape(NSUB, KPT, 128)

    g_bf16 = _sc_gather_scatter(x.astype(jnp.bfloat16), idx3d)

    eps = eps_param.reshape(1, 1)
    y, ssum, ssq = _mlp1(eps, x, g_bf16, W1)
    out = _mlp2(y, ssum, ssq, gamma.reshape(1, D), beta.reshape(1, D), W2)
    return out
